# baseline (speedup 1.0000x reference)
import functools

import jax
import jax.numpy as jnp
from jax import lax
from jax.experimental import pallas as pl
from jax.experimental.pallas import tpu as pltpu
from jax.experimental.pallas import tpu_sc as plsc

_V = 1000000
_D = 64
_B = 4096
_S = 200
_NW = 32
_JT = _V // 128
_JTAIL = _V - _JT * 128


def _relayout_call():
    mesh = plsc.VectorSubcoreMesh(core_axis_name="c", subcore_axis_name="s")

    @functools.partial(
        pl.kernel,
        mesh=mesh,
        out_type=jax.ShapeDtypeStruct((_V // 16, 8, 128), jnp.float32),
        scratch_types=[
            pltpu.VMEM((8, 8, 128), jnp.float32),
            pltpu.VMEM((8, 8, 128), jnp.float32),
        ],
        compiler_params=pltpu.CompilerParams(use_tc_tiling_on_sc=True, needs_layout_passes=False),
    )
    def relayout_kernel(tt_hbm, out_hbm, colbuf, rowbuf):
        wid = lax.axis_index("s") * 2 + lax.axis_index("c")

        iota = lax.iota(jnp.int32, 16)
        idx_c8 = iota % 8
        half = iota // 8

        def transpose_block(n_v):
            def body(pp, carry):
                for qg in range(8):
                    v = 2 * pp + qg // 4
                    idx_i = 2 * (qg % 4) + half
                    vals = plsc.load_gather(
                        colbuf, [idx_i, idx_c8, jnp.full((16,), v, jnp.int32)]
                    )
                    rowbuf[pp // 8, pp % 8, pl.ds(qg * 16, 16)] = vals
                return carry

            lax.fori_loop(0, n_v // 2 // 8 * 8, body, 0, unroll=2)

        def process(j, n_v):
            pltpu.sync_copy(tt_hbm.at[:, :, pl.ds(j * 128, n_v)],
                            colbuf if n_v == 128 else colbuf.at[:, :, pl.ds(0, n_v)])
            transpose_block(n_v)
            n_blk = n_v // 2 // 8
            pltpu.sync_copy(rowbuf.at[pl.ds(0, n_blk)] if n_blk < 8 else rowbuf,
                            out_hbm.at[pl.ds(j * 8, n_blk)])

        def body(i, carry):
            j = wid + i * _NW

            @pl.when(j < _JT)
            def _():
                process(j, 128)

            @pl.when(j == _JT)
            def _():
                process(j, _JTAIL)

            return carry

        lax.fori_loop(0, (_JT + _NW) // _NW, body, 0)

    return relayout_kernel


def _gather_call():
    s_chunk = 4
    n_chunks = _S // s_chunk
    bpw = _B // _NW
    mesh = plsc.VectorSubcoreMesh(core_axis_name="c", subcore_axis_name="s")

    @functools.partial(
        pl.kernel,
        mesh=mesh,
        out_type=jax.ShapeDtypeStruct((_S, 8, _NW, 8, 128), jnp.float32),
        scratch_types=[
            pltpu.VMEM((s_chunk, bpw), jnp.int32),
            pltpu.VMEM((s_chunk * bpw, _D), jnp.float32),
            pltpu.VMEM((s_chunk, 8, 8, 128), jnp.float32),
            pltpu.SemaphoreType.DMA,
        ],
        compiler_params=pltpu.CompilerParams(use_tc_tiling_on_sc=False, needs_layout_passes=False),
    )
    def gather_kernel(x_hbm, table_hbm, out_hbm, idx_s, rows_v,
                      out_blk, sem):
        wid = lax.axis_index("s") * 2 + lax.axis_index("c")
        b0 = wid * bpw

        iota = lax.iota(jnp.int32, 16)

        def body(sc, carry):
            pltpu.sync_copy(
                x_hbm.at[pl.ds(sc * s_chunk, s_chunk), pl.ds(b0, bpw)],
                idx_s)

            copies = [
                pltpu.async_copy(
                    table_hbm.at[idx_s.at[s4]],
                    rows_v.at[pl.ds(s4 * bpw, bpw)],
                    sem,
                )
                for s4 in range(s_chunk)
            ]
            for c in copies:
                c.wait()

            def bgbody(bg, c2):
                for s4 in range(s_chunk):
                    ridx = s4 * bpw + bg * 16 + iota
                    for c in range(_D):
                        vals = plsc.load_gather(
                            rows_v, [ridx, jnp.full((16,), c, jnp.int32)])
                        out_blk[s4, c // 8, c % 8,
                                pl.ds(bg * 16, 16)] = vals
                return c2
            lax.fori_loop(0, bpw // 16, bgbody, 0)

            pltpu.sync_copy(
                out_blk,
                out_hbm.at[pl.ds(sc * s_chunk, s_chunk), :, wid])
            return carry

        lax.fori_loop(0, n_chunks, body, 0)

    return gather_kernel


def kernel(x, table):
    tt = table.T.reshape(8, 8, _V)
    t_pairs = _relayout_call()(tt)
    t_flat = t_pairs.reshape(_V, _D)
    xi = x.T.astype(jnp.int32)
    out5 = _gather_call()(xi, t_flat)
    out = out5.transpose(2, 4, 0, 1, 3).reshape(_B, _S, _D)
    return out


# speedup vs baseline: 1.5797x; 1.5797x over previous
import functools

import jax
import jax.numpy as jnp
from jax import lax
from jax.experimental import pallas as pl
from jax.experimental.pallas import tpu as pltpu
from jax.experimental.pallas import tpu_sc as plsc

_V = 1000000
_D = 64
_B = 4096
_S = 200
_NW = 32
_JT = _V // 128
_JTAIL = _V - _JT * 128


def _relayout_call():
    mesh = plsc.VectorSubcoreMesh(core_axis_name="c", subcore_axis_name="s")

    @functools.partial(
        pl.kernel,
        mesh=mesh,
        out_type=jax.ShapeDtypeStruct((_V // 16, 8, 128), jnp.float32),
        scratch_types=[
            pltpu.VMEM((8, 8, 128), jnp.float32),
            pltpu.VMEM((8, 8, 128), jnp.float32),
        ],
        compiler_params=pltpu.CompilerParams(use_tc_tiling_on_sc=True, needs_layout_passes=False),
    )
    def relayout_kernel(tt_hbm, out_hbm, colbuf, rowbuf):
        wid = lax.axis_index("s") * 2 + lax.axis_index("c")

        iota = lax.iota(jnp.int32, 16)
        idx_c8 = iota % 8
        half = iota // 8

        def transpose_block(n_v):
            @plsc.parallel_loop(0, n_v // 2, unroll=4)
            def _(pp):
                for qg in range(8):
                    v = 2 * pp + qg // 4
                    idx_i = 2 * (qg % 4) + half
                    vals = plsc.load_gather(
                        colbuf, [idx_i, idx_c8, jnp.full((16,), v, jnp.int32)]
                    )
                    rowbuf[pp // 8, pp % 8, pl.ds(qg * 16, 16)] = vals

        def process(j, n_v):
            pltpu.sync_copy(tt_hbm.at[:, :, pl.ds(j * 128, n_v)],
                            colbuf if n_v == 128 else colbuf.at[:, :, pl.ds(0, n_v)])
            transpose_block(n_v)
            n_blk = n_v // 2 // 8
            pltpu.sync_copy(rowbuf.at[pl.ds(0, n_blk)] if n_blk < 8 else rowbuf,
                            out_hbm.at[pl.ds(j * 8, n_blk)])

        def body(i, carry):
            j = wid + i * _NW

            @pl.when(j < _JT)
            def _():
                process(j, 128)

            @pl.when(j == _JT)
            def _():
                process(j, _JTAIL)

            return carry

        lax.fori_loop(0, (_JT + _NW) // _NW, body, 0)

    return relayout_kernel


def _gather_call():
    s_chunk = 4
    n_chunks = _S // s_chunk
    bpw = _B // _NW
    mesh = plsc.VectorSubcoreMesh(core_axis_name="c", subcore_axis_name="s")

    @functools.partial(
        pl.kernel,
        mesh=mesh,
        out_type=jax.ShapeDtypeStruct((_S, 8, _NW, 8, 128), jnp.float32),
        scratch_types=[
            pltpu.VMEM((s_chunk, bpw), jnp.int32),
            pltpu.VMEM((s_chunk * bpw, _D), jnp.float32),
            pltpu.VMEM((s_chunk, 8, 8, 128), jnp.float32),
            pltpu.SemaphoreType.DMA,
        ],
        compiler_params=pltpu.CompilerParams(use_tc_tiling_on_sc=False, needs_layout_passes=False),
    )
    def gather_kernel(x_hbm, table_hbm, out_hbm, idx_s, rows_v,
                      out_blk, sem):
        wid = lax.axis_index("s") * 2 + lax.axis_index("c")
        b0 = wid * bpw

        iota = lax.iota(jnp.int32, 16)

        def body(sc, carry):
            pltpu.sync_copy(
                x_hbm.at[pl.ds(sc * s_chunk, s_chunk), pl.ds(b0, bpw)],
                idx_s)

            copies = [
                pltpu.async_copy(
                    table_hbm.at[idx_s.at[s4]],
                    rows_v.at[pl.ds(s4 * bpw, bpw)],
                    sem,
                )
                for s4 in range(s_chunk)
            ]
            for c in copies:
                c.wait()

            @plsc.parallel_loop(0, bpw // 16)
            def _(bg):
                for s4 in range(s_chunk):
                    ridx = s4 * bpw + bg * 16 + iota
                    for c in range(_D):
                        vals = plsc.load_gather(
                            rows_v, [ridx, jnp.full((16,), c, jnp.int32)])
                        out_blk[s4, c // 8, c % 8,
                                pl.ds(bg * 16, 16)] = vals

            pltpu.sync_copy(
                out_blk,
                out_hbm.at[pl.ds(sc * s_chunk, s_chunk), :, wid])
            return carry

        lax.fori_loop(0, n_chunks, body, 0)

    return gather_kernel


def kernel(x, table):
    tt = table.T.reshape(8, 8, _V)
    t_pairs = _relayout_call()(tt)
    t_flat = t_pairs.reshape(_V, _D)
    xi = x.T.astype(jnp.int32)
    out5 = _gather_call()(xi, t_flat)
    out = out5.transpose(2, 4, 0, 1, 3).reshape(_B, _S, _D)
    return out


# speedup vs baseline: 2.6395x; 1.6709x over previous
import functools

import jax
import jax.numpy as jnp
from jax import lax
from jax.experimental import pallas as pl
from jax.experimental.pallas import tpu as pltpu
from jax.experimental.pallas import tpu_sc as plsc

_NUM_WORKERS = 32
_IDX_MINOR = 128
_K = 5


def _gather_call(n_rows, d):
    chunk = _K * _IDX_MINOR
    rows_per_worker = n_rows // _NUM_WORKERS
    n_chunks = rows_per_worker // chunk
    idx_rows_per_worker = rows_per_worker // _IDX_MINOR

    mesh = plsc.VectorSubcoreMesh(core_axis_name="c", subcore_axis_name="s")

    @functools.partial(
        pl.kernel,
        mesh=mesh,
        out_type=jax.ShapeDtypeStruct((n_rows, d), jnp.float32),
        scratch_types=[
            pltpu.VMEM((idx_rows_per_worker, _IDX_MINOR), jnp.int32),
            pltpu.VMEM((chunk, d), jnp.float32),
            pltpu.VMEM((chunk, d), jnp.float32),
            pltpu.SemaphoreType.DMA,
            pltpu.SemaphoreType.DMA,
            pltpu.SemaphoreType.DMA,
            pltpu.SemaphoreType.DMA,
        ],
        compiler_params=pltpu.CompilerParams(use_tc_tiling_on_sc=False),
    )
    def gather_kernel(idx_hbm, table_hbm, out_hbm, idx_v, rows0, rows1,
                      gs0, gs1, ss0, ss1):
        wid = lax.axis_index("s") * 2 + lax.axis_index("c")
        idx_row_base = wid * idx_rows_per_worker
        out_base = wid * rows_per_worker
        row_bufs = (rows0, rows1)
        gsems = (gs0, gs1)
        ssems = (ss0, ss1)

        def fire_gather(ci, b):
            for j in range(_K):
                pltpu.async_copy(
                    table_hbm.at[idx_v.at[ci * _K + j]],
                    row_bufs[b].at[pl.ds(j * _IDX_MINOR, _IDX_MINOR)],
                    gsems[b],
                )

        def wait_gather(b):
            pltpu.make_async_copy(
                table_hbm.at[pl.ds(0, chunk)], row_bufs[b], gsems[b]
            ).wait()

        def fire_store(ci, b):
            pltpu.async_copy(
                row_bufs[b],
                out_hbm.at[pl.ds(out_base + ci * chunk, chunk)],
                ssems[b],
            )

        def wait_store(b):
            pltpu.make_async_copy(
                row_bufs[b], out_hbm.at[pl.ds(out_base, chunk)], ssems[b]
            ).wait()

        pltpu.sync_copy(idx_hbm.at[pl.ds(idx_row_base, idx_rows_per_worker)],
                        idx_v)
        fire_gather(0, 0)
        fire_gather(1, 1)

        def body(p, carry):
            i = 2 * p
            for b in range(2):
                wait_gather(b)
                fire_store(i + b, b)
            for b in range(2):
                wait_store(b)
                fire_gather(i + b + 2, b)
            return carry

        lax.fori_loop(0, (n_chunks - 2) // 2, body, 0)

        for b in range(2):
            wait_gather(b)
            fire_store(n_chunks - 2 + b, b)
        for b in range(2):
            wait_store(b)

    return gather_kernel


def kernel(x, table):
    b, s = x.shape
    n = b * s
    d = table.shape[1]
    idx = x.reshape(n // _IDX_MINOR, _IDX_MINOR).astype(jnp.int32)
    out = _gather_call(n, d)(idx, table)
    return out.reshape(b, s, d)
